# Initial kernel scaffold; baseline (speedup 1.0000x reference)
import jax
import jax.numpy as jnp
from jax.experimental import pallas as pl


def kernel(input_ids, table):
    raise NotImplementedError("write your pallas kernel here")


# speedup vs baseline: 1.9095x; 1.9095x over previous
import functools

import jax
import jax.numpy as jnp
from jax import lax
from jax.experimental import pallas as pl
from jax.experimental.pallas import tpu as pltpu
from jax.experimental.pallas import tpu_sc as plsc

_D = 768
_NC = 2
_NS = 16
_NW = _NC * _NS
_CHUNK = 32
_NBUF = 4


@functools.lru_cache(maxsize=None)
def _embed_gather(total: int):
    per_w = total // _NW
    nchunk = per_w // _CHUNK
    ngroup = nchunk // _NBUF
    assert per_w * _NW == total and nchunk * _CHUNK == per_w
    assert ngroup * _NBUF == nchunk and ngroup >= 2

    mesh = plsc.VectorSubcoreMesh(
        core_axis_name="c", subcore_axis_name="s",
        num_cores=_NC, num_subcores=_NS)
    scratch = [pltpu.VMEM((nchunk, _CHUNK), jnp.int32)]
    scratch += [pltpu.VMEM((_CHUNK, _D), jnp.float32) for _ in range(_NBUF)]
    scratch += [pltpu.SemaphoreType.DMA for _ in range(2 * _NBUF)]

    @functools.partial(
        pl.kernel,
        mesh=mesh,
        out_type=jax.ShapeDtypeStruct((_NW, per_w, _D), jnp.float32),
        scratch_types=scratch,
    )
    def k(table_hbm, idx_hbm, out_hbm, idx_v, *bufs_and_sems):
        bufs = bufs_and_sems[:_NBUF]
        sem_in = bufs_and_sems[_NBUF:2 * _NBUF]
        sem_out = bufs_and_sems[2 * _NBUF:]
        wid = lax.axis_index("s") * _NC + lax.axis_index("c")

        pltpu.sync_copy(idx_hbm.at[wid], idx_v)

        def start_in(c, b):
            pltpu.make_async_copy(
                table_hbm.at[idx_v.at[c]], bufs[b], sem_in[b]).start()

        def wait_in(c, b):
            pltpu.make_async_copy(
                table_hbm.at[idx_v.at[c]], bufs[b], sem_in[b]).wait()

        def out_slice(c):
            return out_hbm.at[wid, pl.ds(c * _CHUNK, _CHUNK)]

        def start_out(c, b):
            pltpu.make_async_copy(bufs[b], out_slice(c), sem_out[b]).start()

        def wait_out(c, b):
            pltpu.make_async_copy(bufs[b], out_slice(c), sem_out[b]).wait()

        def step(c, b, do_wait_out, do_start_in):
            wait_in(c, b)
            start_out(c, b)
            if do_start_in:
                if do_wait_out:
                    wait_out(c - 2, (b + 2) % _NBUF)
                start_in(c + 2, (b + 2) % _NBUF)

        start_in(0, 0)
        start_in(1, 1)

        for b in range(_NBUF):
            step(b, b, do_wait_out=(b >= 2), do_start_in=(b + 2 < nchunk))

        def body(i, carry):
            c0 = i * _NBUF
            for b in range(_NBUF):
                step(c0 + b, b, True, True)
            return carry
        if ngroup > 2:
            lax.fori_loop(1, ngroup - 1, body, 0)

        c0 = (ngroup - 1) * _NBUF
        for b in range(_NBUF):
            c = c0 + b
            step(c, b, do_wait_out=True, do_start_in=(c + 2 < nchunk))

        for c in range(nchunk - _NBUF, nchunk):
            wait_out(c, c % _NBUF)

    return k


def kernel(input_ids, table):
    b, l = input_ids.shape
    total = b * l
    idx3 = input_ids.reshape(_NW, total // (_NW * _CHUNK), _CHUNK)
    idx3 = idx3.astype(jnp.int32)
    out = _embed_gather(total)(table.astype(jnp.float32), idx3)
    return out.reshape(b, l, _D)
